# baseline (speedup 1.0000x reference)
import functools

import jax
import jax.numpy as jnp
from jax import lax
from jax.experimental import pallas as pl
from jax.experimental.pallas import tpu as pltpu
from jax.experimental.pallas import tpu_sc as plsc

_NW = 32
_IDX_CHUNK = 128


@jax.jit
def kernel(colony_ids, embedding):
    B = colony_ids.shape[0]
    V, D = embedding.shape
    table2 = embedding.T
    b_per_w = B // _NW
    n_chunks = b_per_w // _IDX_CHUNK

    mesh = plsc.VectorSubcoreMesh(core_axis_name="c", subcore_axis_name="s")

    @functools.partial(
        pl.kernel,
        mesh=mesh,
        out_type=jax.ShapeDtypeStruct((D, B), jnp.float32),
        scratch_types=[
            pltpu.VMEM((b_per_w,), jnp.int32),
            pltpu.VMEM((D, b_per_w), jnp.float32),
            pltpu.SemaphoreType.DMA,
        ],
        compiler_params=pltpu.CompilerParams(
            use_tc_tiling_on_sc=False, needs_layout_passes=False
        ),
    )
    def _gather(table_hbm, idx_hbm, out_hbm, idx_v, col_v, sem):
        wid = lax.axis_index("s") * 2 + lax.axis_index("c")
        base = wid * b_per_w
        pltpu.sync_copy(idx_hbm.at[pl.ds(base, b_per_w)], idx_v)

        copies = []
        for d in range(D):
            for g in range(n_chunks):
                o = g * _IDX_CHUNK
                copies.append(
                    pltpu.async_copy(
                        table_hbm.at[d].at[idx_v.at[pl.ds(o, _IDX_CHUNK)]],
                        col_v.at[d].at[pl.ds(o, _IDX_CHUNK)],
                        sem,
                    )
                )
        for c in copies:
            c.wait()

        pltpu.sync_copy(col_v, out_hbm.at[:, pl.ds(base, b_per_w)])

    return _gather(table2, colony_ids).T
